# Initial kernel scaffold; baseline (speedup 1.0000x reference)
import functools

import jax
import jax.numpy as jnp
from jax import lax
from jax.experimental import pallas as pl
from jax.experimental.pallas import tpu as pltpu
from jax.experimental.pallas import tpu_sc as plsc

ADIM_PAD = 8


def _precompute_body(hidden_ref, rela_ref, ws_ref, wr_ref, wq_ref, bias_ref,
                     as_ref, ar_ref, aq_ref):
  as_ref[...] = jnp.dot(hidden_ref[...], ws_ref[...],
                        preferred_element_type=jnp.float32)
  ar_ref[...] = jnp.dot(rela_ref[...], wr_ref[...],
                        preferred_element_type=jnp.float32)
  aq_ref[...] = jnp.dot(rela_ref[...], wq_ref[...],
                        preferred_element_type=jnp.float32) + bias_ref[...]


def _final_body(agg_ref, wh_ref, out_ref):
  out_ref[...] = jnp.dot(agg_ref[0] + agg_ref[1], wh_ref[...],
                         preferred_element_type=jnp.float32)


def _make_sc_kernel(E, N, V, D, B, n_workers):
  CH = 80
  EPT = E // n_workers
  NCHUNK = EPT // CH
  NC = 2
  NS = n_workers // NC
  ROWS_PER_SUB = N // NS
  ZROWS = 125
  NZ = ROWS_PER_SUB // ZROWS
  NV = D // 16

  mesh = plsc.VectorSubcoreMesh(core_axis_name="c", subcore_axis_name="s")

  @functools.partial(
      pl.kernel,
      mesh=mesh,
      out_type=[
          jax.ShapeDtypeStruct((E, D), jnp.float32),
          jax.ShapeDtypeStruct((E,), jnp.float32),
          jax.ShapeDtypeStruct((E,), jnp.float32),
          jax.ShapeDtypeStruct((NC, N, D), jnp.float32),
      ],
      scratch_types=[
          pltpu.VMEM((CH,), jnp.int32),
          pltpu.VMEM((CH,), jnp.int32),
          pltpu.VMEM((CH,), jnp.int32),
          pltpu.VMEM((CH,), jnp.int32),
          pltpu.VMEM((CH,), jnp.int32),
          pltpu.VMEM((CH, 128), jnp.float32),
          pltpu.VMEM((CH, 128), jnp.float32),
          pltpu.VMEM((CH, ADIM_PAD), jnp.float32),
          pltpu.VMEM((CH, ADIM_PAD), jnp.float32),
          pltpu.VMEM((CH, ADIM_PAD), jnp.float32),
          pltpu.VMEM((CH,), jnp.float32),
          pltpu.VMEM((CH,), jnp.float32),
          pltpu.VMEM((16,), jnp.float32),
          pltpu.VMEM((125, 128), jnp.float32),
          pltpu.VMEM_SHARED((N, 128), jnp.float32),
      ],
  )
  def sc_kernel(sub_hbm, rel_hbm, ridx_hbm, obj_hbm, qrel_hbm,
                hidden_hbm, rela_hbm, as_hbm, ar_hbm, aq_hbm, wb_hbm,
                msg_out, alpha_out, at_out, agg_out,
                sub_v, rel_v, ridx_v, obj_v, qr_v,
                hs_v, hr_v, as_v, ar_v, aq_v,
                al_v, at_v, wb_v, z_v, agg_sh):
    cid = lax.axis_index("c")
    sid = lax.axis_index("s")
    wid = sid * NC + cid

    pltpu.sync_copy(wb_hbm, wb_v)

    def zbody(i, _):
      r = i // NV
      j = i % NV
      z_v[r, pl.ds(j * 16, 16)] = jnp.zeros((16,), jnp.float32)
      return 0
    lax.fori_loop(0, ZROWS * NV, zbody, 0)
    for j in range(NZ):
      pltpu.sync_copy(z_v, agg_sh.at[pl.ds(sid * ROWS_PER_SUB + j * ZROWS,
                                           ZROWS)])
    plsc.subcore_barrier()

    w0 = wb_v[0]
    w1 = wb_v[1]
    w2 = wb_v[2]
    w3 = wb_v[3]
    w4 = wb_v[4]
    bias = wb_v[8]
    wk = (w0, w1, w2, w3, w4)

    def chunk_body(c, _):
      base = wid * EPT + c * CH
      pltpu.sync_copy(sub_hbm.at[pl.ds(base, CH)], sub_v)
      pltpu.sync_copy(rel_hbm.at[pl.ds(base, CH)], rel_v)
      pltpu.sync_copy(ridx_hbm.at[pl.ds(base, CH)], ridx_v)
      pltpu.sync_copy(obj_hbm.at[pl.ds(base, CH)], obj_v)
      pltpu.sync_copy(hidden_hbm.at[sub_v], hs_v)
      pltpu.sync_copy(rela_hbm.at[rel_v], hr_v)
      pltpu.sync_copy(qrel_hbm.at[ridx_v], qr_v)
      pltpu.sync_copy(as_hbm.at[sub_v], as_v)
      pltpu.sync_copy(ar_hbm.at[rel_v], ar_v)
      pltpu.sync_copy(aq_hbm.at[qr_v], aq_v)

      lane = lax.iota(jnp.int32, 16)
      for g in range(CH // 16):
        rows = lane + g * 16
        acc = jnp.zeros((16,), jnp.float32)
        for k in range(5):
          cols = jnp.full((16,), k, jnp.int32)
          sv = plsc.load_gather(as_v, [rows, cols])
          rv = plsc.load_gather(ar_v, [rows, cols])
          qv = plsc.load_gather(aq_v, [rows, cols])
          acc = acc + wk[k] * jnp.maximum(sv + rv + qv, 0.0)
      at = acc + bias
      al = 1.0 / (1.0 + jnp.exp(-at))
      at_v[pl.ds(g * 16, 16)] = at
      al_v[pl.ds(g * 16, 16)] = al

      def mbody(e, _):
        a = al_v[e]
        for j in range(NV):
          hs = hs_v[e, pl.ds(j * 16, 16)]
          hr = hr_v[e, pl.ds(j * 16, 16)]
          hs_v[e, pl.ds(j * 16, 16)] = (hs + hr) * a
        return 0
      lax.fori_loop(0, CH, mbody, 0)

      pltpu.sync_copy(hs_v, msg_out.at[pl.ds(base, CH)])
      pltpu.sync_copy(al_v, alpha_out.at[pl.ds(base, CH)])
      pltpu.sync_copy(at_v, at_out.at[pl.ds(base, CH)])
      pltpu.sync_copy(hs_v, agg_sh.at[obj_v], add=True)
      return 0
    lax.fori_loop(0, NCHUNK, chunk_body, 0)

    plsc.subcore_barrier()
    for j in range(NZ):
      rows = pl.ds(sid * ROWS_PER_SUB + j * ZROWS, ZROWS)
      pltpu.sync_copy(agg_sh.at[rows], agg_out.at[cid, rows])

  return sc_kernel


def kernel(q_sub, q_rel, hidden, edges, n_node, rela_embed, Ws_attn, Wr_attn,
           Wqr_attn, Wqr_bias, w_alpha, w_alpha_bias, W_h):
  N, D = hidden.shape
  V = rela_embed.shape[0]
  E = edges.shape[0]
  B = q_rel.shape[0]
  n_workers = 32

  sub = edges[:, 4].astype(jnp.int32)
  rel = edges[:, 2].astype(jnp.int32)
  obj = edges[:, 5].astype(jnp.int32)
  r_idx = edges[:, 0].astype(jnp.int32)
  q_rel32 = q_rel.astype(jnp.int32)

  def pad_w(w):
    return jnp.pad(w, ((0, 0), (0, ADIM_PAD - w.shape[1])))

  bias_pad = jnp.pad(Wqr_bias, (0, ADIM_PAD - Wqr_bias.shape[0]))[None, :]
  vpad = (-V) % 8
  rela_pad = jnp.pad(rela_embed, ((0, vpad), (0, 0)))
  Vp = V + vpad

  a_s, a_r, a_q = pl.pallas_call(
      _precompute_body,
      out_shape=[
          jax.ShapeDtypeStruct((N, ADIM_PAD), jnp.float32),
          jax.ShapeDtypeStruct((Vp, ADIM_PAD), jnp.float32),
          jax.ShapeDtypeStruct((Vp, ADIM_PAD), jnp.float32),
      ],
  )(hidden, rela_pad, pad_w(Ws_attn), pad_w(Wr_attn), pad_w(Wqr_attn),
    bias_pad)

  wb = jnp.zeros((16,), jnp.float32)
  wb = wb.at[0:5].set(w_alpha[:, 0])
  wb = wb.at[8].set(w_alpha_bias[0])

  sc_kernel = _make_sc_kernel(E, N, Vp, D, B, n_workers)
  message, alpha, alpha_temp, aggs = sc_kernel(
      sub, rel, r_idx, obj, q_rel32, hidden, rela_embed, a_s, a_r, a_q, wb)

  hidden_new = pl.pallas_call(
      _final_body,
      out_shape=jax.ShapeDtypeStruct((N, D), jnp.float32),
  )(aggs, W_h)

  return (hidden_new, alpha[:, None], message, obj, alpha_temp[:, None])


# speedup vs baseline: 1.6762x; 1.6762x over previous
import functools

import jax
import jax.numpy as jnp
from jax import lax
from jax.experimental import pallas as pl
from jax.experimental.pallas import tpu as pltpu
from jax.experimental.pallas import tpu_sc as plsc

ADIM = 5
ADIM_PAD = 8


def _precompute_body(hidden_ref, rela_ref, ws_ref, wr_ref, wq_ref, bias_ref,
                     as_ref, ar_ref, aq_ref):
  dn = (((0,), (1,)), ((), ()))
  as_ref[...] = lax.dot_general(ws_ref[...], hidden_ref[...], dn,
                                preferred_element_type=jnp.float32)
  ar_ref[...] = lax.dot_general(wr_ref[...], rela_ref[...], dn,
                                preferred_element_type=jnp.float32)
  aq_ref[...] = lax.dot_general(wq_ref[...], rela_ref[...], dn,
                                preferred_element_type=jnp.float32) + bias_ref[...]


def _final_body(agg_ref, wh_ref, out_ref):
  n = out_ref.shape[0]
  out_ref[...] = jnp.dot(agg_ref[0, :n, :] + agg_ref[1, :n, :], wh_ref[...],
                         preferred_element_type=jnp.float32)


def _make_sc_kernel(E, N, D, n_workers):
  CH = 80
  EPT = E // n_workers
  NCHUNK = EPT // CH
  NC = 2
  NS = n_workers // NC
  NPAD = -(-N // (NS * 40)) * NS * 40
  ROWS_PER_SUB = NPAD // NS
  ZROWS = ROWS_PER_SUB // 5
  NZ = 5
  NV = D // 16

  mesh = plsc.VectorSubcoreMesh(core_axis_name="c", subcore_axis_name="s")

  @functools.partial(
      pl.kernel,
      mesh=mesh,
      out_type=[
          jax.ShapeDtypeStruct((E, D), jnp.float32),
          jax.ShapeDtypeStruct((E,), jnp.float32),
          jax.ShapeDtypeStruct((E,), jnp.float32),
          jax.ShapeDtypeStruct((NC, NPAD, D), jnp.float32),
      ],
      scratch_types=[
          pltpu.VMEM((CH,), jnp.int32),
          pltpu.VMEM((CH,), jnp.int32),
          pltpu.VMEM((CH,), jnp.int32),
          pltpu.VMEM((CH,), jnp.int32),
          pltpu.VMEM((CH,), jnp.int32),
          pltpu.VMEM((CH, 128), jnp.float32),
          pltpu.VMEM((CH, 128), jnp.float32),
          pltpu.VMEM((3 * ADIM, CH), jnp.float32),
          pltpu.VMEM((CH,), jnp.float32),
          pltpu.VMEM((CH,), jnp.float32),
          pltpu.VMEM((16,), jnp.float32),
          pltpu.VMEM((ZROWS, 128), jnp.float32),
          pltpu.VMEM_SHARED((NPAD, 128), jnp.float32),
      ],
  )
  def sc_kernel(sub_hbm, rel_hbm, ridx_hbm, obj_hbm, qrel_hbm,
                hidden_hbm, rela_hbm,
                s0, s1, s2, s3, s4, r0, r1, r2, r3, r4, q0, q1, q2, q3, q4,
                wb_hbm,
                msg_out, alpha_out, at_out, agg_out,
                sub_v, rel_v, ridx_v, obj_v, qr_v,
                hs_v, hr_v, att_v, al_v, at_v, wb_v, z_v, agg_sh):
    cid = lax.axis_index("c")
    sid = lax.axis_index("s")
    wid = sid * NC + cid
    scols = (s0, s1, s2, s3, s4)
    rcols = (r0, r1, r2, r3, r4)
    qcols = (q0, q1, q2, q3, q4)

    pltpu.sync_copy(wb_hbm, wb_v)

    def zbody(i, _):
      r = i // NV
      j = i % NV
      z_v[r, pl.ds(j * 16, 16)] = jnp.zeros((16,), jnp.float32)
      return 0
    lax.fori_loop(0, ZROWS * NV, zbody, 0)
    for j in range(NZ):
      pltpu.sync_copy(z_v, agg_sh.at[pl.ds(sid * ROWS_PER_SUB + j * ZROWS,
                                           ZROWS)])
    plsc.subcore_barrier()

    wv = wb_v[...]
    bias = wv[8]
    wk = (wv[0], wv[1], wv[2], wv[3], wv[4])

    def chunk_body(c, _):
      base = wid * EPT + c * CH
      pltpu.sync_copy(sub_hbm.at[pl.ds(base, CH)], sub_v)
      pltpu.sync_copy(rel_hbm.at[pl.ds(base, CH)], rel_v)
      pltpu.sync_copy(ridx_hbm.at[pl.ds(base, CH)], ridx_v)
      pltpu.sync_copy(obj_hbm.at[pl.ds(base, CH)], obj_v)
      pltpu.sync_copy(hidden_hbm.at[sub_v], hs_v)
      pltpu.sync_copy(rela_hbm.at[rel_v], hr_v)
      pltpu.sync_copy(qrel_hbm.at[ridx_v], qr_v)
      for k in range(ADIM):
        pltpu.sync_copy(scols[k].at[sub_v], att_v.at[k])
        pltpu.sync_copy(rcols[k].at[rel_v], att_v.at[ADIM + k])
        pltpu.sync_copy(qcols[k].at[qr_v], att_v.at[2 * ADIM + k])

      for g in range(CH // 16):
        dg = pl.ds(g * 16, 16)
        acc = jnp.zeros((16,), jnp.float32)
        for k in range(ADIM):
          t = att_v[k, dg] + att_v[ADIM + k, dg] + att_v[2 * ADIM + k, dg]
          acc = acc + wk[k] * jnp.maximum(t, 0.0)
        at = acc + bias
        al = 1.0 / (1.0 + jnp.exp(-at))
        at_v[dg] = at
        al_v[dg] = al

      def mbody(g, _):
        a16 = al_v[pl.ds(g * 16, 16)]
        for i in range(16):
          e = g * 16 + i
          a = a16[i]
          for j in range(NV):
            hs = hs_v[e, pl.ds(j * 16, 16)]
            hr = hr_v[e, pl.ds(j * 16, 16)]
            hs_v[e, pl.ds(j * 16, 16)] = (hs + hr) * a
        return 0
      lax.fori_loop(0, CH // 16, mbody, 0)

      pltpu.sync_copy(hs_v, msg_out.at[pl.ds(base, CH)])
      pltpu.sync_copy(al_v, alpha_out.at[pl.ds(base, CH)])
      pltpu.sync_copy(at_v, at_out.at[pl.ds(base, CH)])
      pltpu.sync_copy(hs_v, agg_sh.at[obj_v], add=True)
      return 0
    lax.fori_loop(0, NCHUNK, chunk_body, 0)

    plsc.subcore_barrier()
    for j in range(NZ):
      rows = pl.ds(sid * ROWS_PER_SUB + j * ZROWS, ZROWS)
      pltpu.sync_copy(agg_sh.at[rows], agg_out.at[cid, rows])

  return sc_kernel


def kernel(q_sub, q_rel, hidden, edges, n_node, rela_embed, Ws_attn, Wr_attn,
           Wqr_attn, Wqr_bias, w_alpha, w_alpha_bias, W_h):
  N, D = hidden.shape
  V = rela_embed.shape[0]
  E = edges.shape[0]
  n_workers = 32

  sub = edges[:, 4].astype(jnp.int32)
  rel = edges[:, 2].astype(jnp.int32)
  obj = edges[:, 5].astype(jnp.int32)
  r_idx = edges[:, 0].astype(jnp.int32)
  q_rel32 = q_rel.astype(jnp.int32)

  def pad_w(w):
    return jnp.pad(w, ((0, 0), (0, ADIM_PAD - w.shape[1])))

  bias_pad = jnp.pad(Wqr_bias, (0, ADIM_PAD - Wqr_bias.shape[0]))[:, None]
  vpad = (-V) % 8
  rela_pad = jnp.pad(rela_embed, ((0, vpad), (0, 0)))
  Vp = V + vpad

  a_sT, a_rT, a_qT = pl.pallas_call(
      _precompute_body,
      out_shape=[
          jax.ShapeDtypeStruct((ADIM_PAD, N), jnp.float32),
          jax.ShapeDtypeStruct((ADIM_PAD, Vp), jnp.float32),
          jax.ShapeDtypeStruct((ADIM_PAD, Vp), jnp.float32),
      ],
  )(hidden, rela_pad, pad_w(Ws_attn), pad_w(Wr_attn), pad_w(Wqr_attn),
    bias_pad)

  wb = jnp.zeros((16,), jnp.float32)
  wb = wb.at[0:5].set(w_alpha[:, 0])
  wb = wb.at[8].set(w_alpha_bias[0])

  sc_kernel = _make_sc_kernel(E, N, D, n_workers)
  cols = ([a_sT[k] for k in range(ADIM)] + [a_rT[k] for k in range(ADIM)]
          + [a_qT[k] for k in range(ADIM)])
  message, alpha, alpha_temp, aggs = sc_kernel(
      sub, rel, r_idx, obj, q_rel32, hidden, rela_embed, *cols, wb)

  hidden_new = pl.pallas_call(
      _final_body,
      out_shape=jax.ShapeDtypeStruct((N, D), jnp.float32),
  )(aggs, W_h)

  return (hidden_new, alpha[:, None], message, obj, alpha_temp[:, None])


# speedup vs baseline: 1.8379x; 1.0964x over previous
import functools

import jax
import jax.numpy as jnp
from jax import lax
from jax.experimental import pallas as pl
from jax.experimental.pallas import tpu as pltpu
from jax.experimental.pallas import tpu_sc as plsc

ADIM = 5
ADIM_PAD = 8


def _precompute_body(hidden_ref, rela_ref, ws_ref, wr_ref, wq_ref, bias_ref,
                     as_ref, ar_ref, aq_ref):
  dn = (((0,), (1,)), ((), ()))
  as_ref[...] = lax.dot_general(ws_ref[...], hidden_ref[...], dn,
                                preferred_element_type=jnp.float32)
  ar_ref[...] = lax.dot_general(wr_ref[...], rela_ref[...], dn,
                                preferred_element_type=jnp.float32)
  aq_ref[...] = lax.dot_general(wq_ref[...], rela_ref[...], dn,
                                preferred_element_type=jnp.float32) + bias_ref[...]


def _final_body(agg_ref, wh_ref, out_ref):
  n = out_ref.shape[0]
  out_ref[...] = jnp.dot(agg_ref[0, :n, :] + agg_ref[1, :n, :], wh_ref[...],
                         preferred_element_type=jnp.float32)


def _make_sc_kernel(E, N, D, n_workers):
  CH = 80
  EPT = E // n_workers
  NCHUNK = EPT // CH
  NC = 2
  NS = n_workers // NC
  NPAD = -(-N // (NS * 40)) * NS * 40
  ROWS_PER_SUB = NPAD // NS
  ZROWS = ROWS_PER_SUB // 5
  NZ = 5
  NV = D // 16

  mesh = plsc.VectorSubcoreMesh(core_axis_name="c", subcore_axis_name="s")

  @functools.partial(
      pl.kernel,
      mesh=mesh,
      out_type=[
          jax.ShapeDtypeStruct((E, D), jnp.float32),
          jax.ShapeDtypeStruct((E,), jnp.float32),
          jax.ShapeDtypeStruct((E,), jnp.float32),
          jax.ShapeDtypeStruct((NC, NPAD, D), jnp.float32),
      ],
      scratch_types=[
          pltpu.VMEM((CH,), jnp.int32),
          pltpu.VMEM((CH,), jnp.int32),
          pltpu.VMEM((CH,), jnp.int32),
          pltpu.VMEM((CH,), jnp.int32),
          pltpu.VMEM((CH,), jnp.int32),
          pltpu.VMEM((CH, 128), jnp.float32),
          pltpu.VMEM((CH, 128), jnp.float32),
          pltpu.VMEM((3 * ADIM, CH), jnp.float32),
          pltpu.VMEM((CH,), jnp.float32),
          pltpu.VMEM((CH,), jnp.float32),
          pltpu.VMEM((16,), jnp.float32),
          pltpu.VMEM((ZROWS, 128), jnp.float32),
          pltpu.VMEM_SHARED((NPAD, 128), jnp.float32),
          pltpu.SemaphoreType.DMA,
          pltpu.SemaphoreType.DMA,
          pltpu.SemaphoreType.DMA,
          pltpu.SemaphoreType.DMA,
      ],
  )
  def sc_kernel(sub_hbm, rel_hbm, ridx_hbm, obj_hbm, qrel_hbm,
                hidden_hbm, rela_hbm,
                s0, s1, s2, s3, s4, r0, r1, r2, r3, r4, q0, q1, q2, q3, q4,
                wb_hbm,
                msg_out, alpha_out, at_out, agg_out,
                sub_v, rel_v, ridx_v, obj_v, qr_v,
                hs_v, hr_v, att_v, al_v, at_v, wb_v, z_v, agg_sh,
                sem_idx, sem_gat, sem_qr, sem_wr):
    cid = lax.axis_index("c")
    sid = lax.axis_index("s")
    wid = sid * NC + cid
    scols = (s0, s1, s2, s3, s4)
    rcols = (r0, r1, r2, r3, r4)
    qcols = (q0, q1, q2, q3, q4)

    pltpu.sync_copy(wb_hbm, wb_v)

    def zbody(i, _):
      r = i // NV
      j = i % NV
      z_v[r, pl.ds(j * 16, 16)] = jnp.zeros((16,), jnp.float32)
      return 0
    lax.fori_loop(0, ZROWS * NV, zbody, 0)
    for j in range(NZ):
      pltpu.sync_copy(z_v, agg_sh.at[pl.ds(sid * ROWS_PER_SUB + j * ZROWS,
                                           ZROWS)])
    plsc.subcore_barrier()

    wv = wb_v[...]
    bias = wv[8]
    wk = (wv[0], wv[1], wv[2], wv[3], wv[4])

    def chunk_body(c, _):
      base = wid * EPT + c * CH
      idx_cps = [
          pltpu.async_copy(sub_hbm.at[pl.ds(base, CH)], sub_v, sem_idx),
          pltpu.async_copy(rel_hbm.at[pl.ds(base, CH)], rel_v, sem_idx),
          pltpu.async_copy(ridx_hbm.at[pl.ds(base, CH)], ridx_v, sem_idx),
          pltpu.async_copy(obj_hbm.at[pl.ds(base, CH)], obj_v, sem_idx),
      ]
      for cp in idx_cps:
        cp.wait()
      pltpu.sync_copy(hidden_hbm.at[sub_v], hs_v)
      pltpu.sync_copy(rela_hbm.at[rel_v], hr_v)
      pltpu.sync_copy(qrel_hbm.at[ridx_v], qr_v)
      for k in range(ADIM):
        pltpu.sync_copy(scols[k].at[sub_v], att_v.at[k])
        pltpu.sync_copy(rcols[k].at[rel_v], att_v.at[ADIM + k])
        pltpu.sync_copy(qcols[k].at[qr_v], att_v.at[2 * ADIM + k])

      for g in range(CH // 16):
        dg = pl.ds(g * 16, 16)
        acc = jnp.zeros((16,), jnp.float32)
        for k in range(ADIM):
          t = att_v[k, dg] + att_v[ADIM + k, dg] + att_v[2 * ADIM + k, dg]
          acc = acc + wk[k] * jnp.maximum(t, 0.0)
        at = acc + bias
        al = 1.0 / (1.0 + jnp.exp(-at))
        at_v[dg] = at
        al_v[dg] = al

      def mbody(g, _):
        a16 = al_v[pl.ds(g * 16, 16)]
        for i in range(16):
          e = g * 16 + i
          a = a16[i]
          for j in range(NV):
            hs = hs_v[e, pl.ds(j * 16, 16)]
            hr = hr_v[e, pl.ds(j * 16, 16)]
            hs_v[e, pl.ds(j * 16, 16)] = (hs + hr) * a
        return 0
      lax.fori_loop(0, CH // 16, mbody, 0)

      wr_cps = [
          pltpu.async_copy(hs_v, msg_out.at[pl.ds(base, CH)], sem_wr),
          pltpu.async_copy(al_v, alpha_out.at[pl.ds(base, CH)], sem_wr),
          pltpu.async_copy(at_v, at_out.at[pl.ds(base, CH)], sem_wr),
      ]
      pltpu.sync_copy(hs_v, agg_sh.at[obj_v], add=True)
      for cp in wr_cps:
        cp.wait()
      return 0
    lax.fori_loop(0, NCHUNK, chunk_body, 0)

    plsc.subcore_barrier()
    for j in range(NZ):
      rows = pl.ds(sid * ROWS_PER_SUB + j * ZROWS, ZROWS)
      pltpu.sync_copy(agg_sh.at[rows], agg_out.at[cid, rows])

  return sc_kernel


def kernel(q_sub, q_rel, hidden, edges, n_node, rela_embed, Ws_attn, Wr_attn,
           Wqr_attn, Wqr_bias, w_alpha, w_alpha_bias, W_h):
  N, D = hidden.shape
  V = rela_embed.shape[0]
  E = edges.shape[0]
  n_workers = 32

  sub = edges[:, 4].astype(jnp.int32)
  rel = edges[:, 2].astype(jnp.int32)
  obj = edges[:, 5].astype(jnp.int32)
  r_idx = edges[:, 0].astype(jnp.int32)
  q_rel32 = q_rel.astype(jnp.int32)

  def pad_w(w):
    return jnp.pad(w, ((0, 0), (0, ADIM_PAD - w.shape[1])))

  bias_pad = jnp.pad(Wqr_bias, (0, ADIM_PAD - Wqr_bias.shape[0]))[:, None]
  vpad = (-V) % 8
  rela_pad = jnp.pad(rela_embed, ((0, vpad), (0, 0)))
  Vp = V + vpad

  a_sT, a_rT, a_qT = pl.pallas_call(
      _precompute_body,
      out_shape=[
          jax.ShapeDtypeStruct((ADIM_PAD, N), jnp.float32),
          jax.ShapeDtypeStruct((ADIM_PAD, Vp), jnp.float32),
          jax.ShapeDtypeStruct((ADIM_PAD, Vp), jnp.float32),
      ],
  )(hidden, rela_pad, pad_w(Ws_attn), pad_w(Wr_attn), pad_w(Wqr_attn),
    bias_pad)

  wb = jnp.zeros((16,), jnp.float32)
  wb = wb.at[0:5].set(w_alpha[:, 0])
  wb = wb.at[8].set(w_alpha_bias[0])

  sc_kernel = _make_sc_kernel(E, N, D, n_workers)
  cols = ([a_sT[k] for k in range(ADIM)] + [a_rT[k] for k in range(ADIM)]
          + [a_qT[k] for k in range(ADIM)])
  message, alpha, alpha_temp, aggs = sc_kernel(
      sub, rel, r_idx, obj, q_rel32, hidden, rela_embed, *cols, wb)

  hidden_new = pl.pallas_call(
      _final_body,
      out_shape=jax.ShapeDtypeStruct((N, D), jnp.float32),
  )(aggs, W_h)

  return (hidden_new, alpha[:, None], message, obj, alpha_temp[:, None])


# speedup vs baseline: 4.1278x; 2.2459x over previous
import functools

import jax
import jax.numpy as jnp
from jax import lax
from jax.experimental import pallas as pl
from jax.experimental.pallas import tpu as pltpu
from jax.experimental.pallas import tpu_sc as plsc

ADIM = 5
ADIM_PAD = 8


def _precompute_body(hidden_ref, rela_ref, ws_ref, wr_ref, wq_ref, bias_ref,
                     as_ref, ar_ref, aq_ref):
  dn = (((0,), (1,)), ((), ()))
  as_ref[...] = lax.dot_general(ws_ref[...], hidden_ref[...], dn,
                                preferred_element_type=jnp.float32)
  ar_ref[...] = lax.dot_general(wr_ref[...], rela_ref[...], dn,
                                preferred_element_type=jnp.float32)
  aq_ref[...] = lax.dot_general(wq_ref[...], rela_ref[...], dn,
                                preferred_element_type=jnp.float32) + bias_ref[...]


def _final_body(agg_ref, wh_ref, out_ref):
  n = out_ref.shape[0]
  out_ref[...] = jnp.dot(agg_ref[0, :n, :] + agg_ref[1, :n, :], wh_ref[...],
                         preferred_element_type=jnp.float32)


def _make_sc_kernel(E, N, D, n_workers):
  CH = 80
  EPT = E // n_workers
  NCHUNK = EPT // CH
  NC = 2
  NS = n_workers // NC
  NPAD = -(-N // (NS * 40)) * NS * 40
  ROWS_PER_SUB = NPAD // NS
  ZROWS = ROWS_PER_SUB // 5
  NZ = 5
  NV = D // 16

  mesh = plsc.VectorSubcoreMesh(core_axis_name="c", subcore_axis_name="s")

  @functools.partial(
      pl.kernel,
      mesh=mesh,
      out_type=[
          jax.ShapeDtypeStruct((E, D), jnp.float32),
          jax.ShapeDtypeStruct((E,), jnp.float32),
          jax.ShapeDtypeStruct((E,), jnp.float32),
          jax.ShapeDtypeStruct((NC, NPAD, D), jnp.float32),
      ],
      scratch_types=[
          pltpu.VMEM((CH,), jnp.int32),
          pltpu.VMEM((CH,), jnp.int32),
          pltpu.VMEM((CH,), jnp.int32),
          pltpu.VMEM((CH,), jnp.int32),
          pltpu.VMEM((CH,), jnp.int32),
          pltpu.VMEM((CH, 128), jnp.float32),
          pltpu.VMEM((CH, 128), jnp.float32),
          pltpu.VMEM((3 * ADIM, CH), jnp.float32),
          pltpu.VMEM((CH,), jnp.float32),
          pltpu.VMEM((CH,), jnp.float32),
          pltpu.VMEM((16,), jnp.float32),
          pltpu.VMEM((ZROWS, 128), jnp.float32),
          pltpu.VMEM_SHARED((NPAD, 128), jnp.float32),
          pltpu.SemaphoreType.DMA,
          pltpu.SemaphoreType.DMA,
          pltpu.SemaphoreType.DMA,
          pltpu.SemaphoreType.DMA,
      ],
  )
  def sc_kernel(sub_hbm, rel_hbm, ridx_hbm, obj_hbm, qrel_hbm,
                hidden_hbm, rela_hbm,
                s0, s1, s2, s3, s4, r0, r1, r2, r3, r4, q0, q1, q2, q3, q4,
                wb_hbm,
                msg_out, alpha_out, at_out, agg_out,
                sub_v, rel_v, ridx_v, obj_v, qr_v,
                hs_v, hr_v, att_v, al_v, at_v, wb_v, z_v, agg_sh,
                sem_idx, sem_gat, sem_qr, sem_wr):
    cid = lax.axis_index("c")
    sid = lax.axis_index("s")
    wid = sid * NC + cid
    scols = (s0, s1, s2, s3, s4)
    rcols = (r0, r1, r2, r3, r4)
    qcols = (q0, q1, q2, q3, q4)

    pltpu.sync_copy(wb_hbm, wb_v)

    def zbody(i, _):
      r = i // NV
      j = i % NV
      z_v[r, pl.ds(j * 16, 16)] = jnp.zeros((16,), jnp.float32)
      return 0
    lax.fori_loop(0, ZROWS * NV, zbody, 0)
    for j in range(NZ):
      pltpu.sync_copy(z_v, agg_sh.at[pl.ds(sid * ROWS_PER_SUB + j * ZROWS,
                                           ZROWS)])
    plsc.subcore_barrier()

    wv = wb_v[...]
    bias = wv[8]
    wk = (wv[0], wv[1], wv[2], wv[3], wv[4])

    def chunk_body(c, _):
      base = wid * EPT + c * CH
      idx_cps = [
          pltpu.async_copy(sub_hbm.at[pl.ds(base, CH)], sub_v, sem_idx),
          pltpu.async_copy(rel_hbm.at[pl.ds(base, CH)], rel_v, sem_idx),
          pltpu.async_copy(ridx_hbm.at[pl.ds(base, CH)], ridx_v, sem_idx),
          pltpu.async_copy(obj_hbm.at[pl.ds(base, CH)], obj_v, sem_idx),
      ]
      for cp in idx_cps:
        cp.wait()
      pltpu.sync_copy(qrel_hbm.at[ridx_v], qr_v)
      g1 = [pltpu.async_copy(hidden_hbm.at[sub_v], hs_v, sem_gat),
            pltpu.async_copy(rela_hbm.at[rel_v], hr_v, sem_gat)]
      for k in range(ADIM):
        g1.append(pltpu.async_copy(scols[k].at[sub_v], att_v.at[k], sem_gat))
        g1.append(pltpu.async_copy(rcols[k].at[rel_v], att_v.at[ADIM + k],
                                   sem_gat))
        g1.append(pltpu.async_copy(qcols[k].at[qr_v], att_v.at[2 * ADIM + k],
                                   sem_gat))
      for cp in g1:
        cp.wait()

      for g in range(CH // 16):
        dg = pl.ds(g * 16, 16)
        acc = jnp.zeros((16,), jnp.float32)
        for k in range(ADIM):
          t = att_v[k, dg] + att_v[ADIM + k, dg] + att_v[2 * ADIM + k, dg]
          acc = acc + wk[k] * jnp.maximum(t, 0.0)
        at = acc + bias
        al = 1.0 / (1.0 + jnp.exp(-at))
        at_v[dg] = at
        al_v[dg] = al

      def mbody(g, _):
        a16 = al_v[pl.ds(g * 16, 16)]
        for i in range(16):
          e = g * 16 + i
          a = a16[i]
          for j in range(NV):
            hs = hs_v[e, pl.ds(j * 16, 16)]
            hr = hr_v[e, pl.ds(j * 16, 16)]
            hs_v[e, pl.ds(j * 16, 16)] = (hs + hr) * a
        return 0
      lax.fori_loop(0, CH // 16, mbody, 0)

      wr_cps = [
          pltpu.async_copy(hs_v, msg_out.at[pl.ds(base, CH)], sem_wr),
          pltpu.async_copy(al_v, alpha_out.at[pl.ds(base, CH)], sem_wr),
          pltpu.async_copy(at_v, at_out.at[pl.ds(base, CH)], sem_wr),
      ]
      pltpu.sync_copy(hs_v, agg_sh.at[obj_v], add=True)
      for cp in wr_cps:
        cp.wait()
      return 0
    lax.fori_loop(0, NCHUNK, chunk_body, 0)

    plsc.subcore_barrier()
    for j in range(NZ):
      rows = pl.ds(sid * ROWS_PER_SUB + j * ZROWS, ZROWS)
      pltpu.sync_copy(agg_sh.at[rows], agg_out.at[cid, rows])

  return sc_kernel


def kernel(q_sub, q_rel, hidden, edges, n_node, rela_embed, Ws_attn, Wr_attn,
           Wqr_attn, Wqr_bias, w_alpha, w_alpha_bias, W_h):
  N, D = hidden.shape
  V = rela_embed.shape[0]
  E = edges.shape[0]
  n_workers = 32

  sub = edges[:, 4].astype(jnp.int32)
  rel = edges[:, 2].astype(jnp.int32)
  obj = edges[:, 5].astype(jnp.int32)
  r_idx = edges[:, 0].astype(jnp.int32)
  q_rel32 = q_rel.astype(jnp.int32)

  def pad_w(w):
    return jnp.pad(w, ((0, 0), (0, ADIM_PAD - w.shape[1])))

  bias_pad = jnp.pad(Wqr_bias, (0, ADIM_PAD - Wqr_bias.shape[0]))[:, None]
  vpad = (-V) % 8
  rela_pad = jnp.pad(rela_embed, ((0, vpad), (0, 0)))
  Vp = V + vpad

  a_sT, a_rT, a_qT = pl.pallas_call(
      _precompute_body,
      out_shape=[
          jax.ShapeDtypeStruct((ADIM_PAD, N), jnp.float32),
          jax.ShapeDtypeStruct((ADIM_PAD, Vp), jnp.float32),
          jax.ShapeDtypeStruct((ADIM_PAD, Vp), jnp.float32),
      ],
  )(hidden, rela_pad, pad_w(Ws_attn), pad_w(Wr_attn), pad_w(Wqr_attn),
    bias_pad)

  wb = jnp.zeros((16,), jnp.float32)
  wb = wb.at[0:5].set(w_alpha[:, 0])
  wb = wb.at[8].set(w_alpha_bias[0])

  sc_kernel = _make_sc_kernel(E, N, D, n_workers)
  cols = ([a_sT[k] for k in range(ADIM)] + [a_rT[k] for k in range(ADIM)]
          + [a_qT[k] for k in range(ADIM)])
  message, alpha, alpha_temp, aggs = sc_kernel(
      sub, rel, r_idx, obj, q_rel32, hidden, rela_embed, *cols, wb)

  hidden_new = pl.pallas_call(
      _final_body,
      out_shape=jax.ShapeDtypeStruct((N, D), jnp.float32),
  )(aggs, W_h)

  return (hidden_new, alpha[:, None], message, obj, alpha_temp[:, None])


# speedup vs baseline: 4.4402x; 1.0757x over previous
import functools

import jax
import jax.numpy as jnp
from jax import lax
from jax.experimental import pallas as pl
from jax.experimental.pallas import tpu as pltpu
from jax.experimental.pallas import tpu_sc as plsc

ADIM = 5
ADIM_PAD = 8


def _precompute_body(hidden_ref, rela_ref, ws_ref, wr_ref, wq_ref, bias_ref,
                     as_ref, ar_ref, aq_ref):
  dn = (((0,), (1,)), ((), ()))
  as_ref[...] = lax.dot_general(ws_ref[...], hidden_ref[...], dn,
                                preferred_element_type=jnp.float32)
  ar_ref[...] = lax.dot_general(wr_ref[...], rela_ref[...], dn,
                                preferred_element_type=jnp.float32)
  aq_ref[...] = lax.dot_general(wq_ref[...], rela_ref[...], dn,
                                preferred_element_type=jnp.float32) + bias_ref[...]


def _final_body(agg_ref, wh_ref, out_ref):
  n = out_ref.shape[0]
  out_ref[...] = jnp.dot(agg_ref[0, :n, :] + agg_ref[1, :n, :], wh_ref[...],
                         preferred_element_type=jnp.float32)


def _make_sc_kernel(E, N, D, n_workers):
  CH = 128
  TCH = E // CH
  NC = 2
  NS = n_workers // NC
  NPAD = -(-N // (NS * 40)) * NS * 40
  ROWS_PER_SUB = NPAD // NS
  ZROWS = ROWS_PER_SUB // 10
  NZ = 10
  NV = D // 16

  mesh = plsc.VectorSubcoreMesh(core_axis_name="c", subcore_axis_name="s")

  @functools.partial(
      pl.kernel,
      mesh=mesh,
      out_type=[
          jax.ShapeDtypeStruct((E, D), jnp.float32),
          jax.ShapeDtypeStruct((E,), jnp.float32),
          jax.ShapeDtypeStruct((E,), jnp.float32),
          jax.ShapeDtypeStruct((NC, NPAD, D), jnp.float32),
      ],
      scratch_types=[
          pltpu.VMEM((CH,), jnp.int32),
          pltpu.VMEM((CH,), jnp.int32),
          pltpu.VMEM((CH,), jnp.int32),
          pltpu.VMEM((CH,), jnp.int32),
          pltpu.VMEM((CH,), jnp.int32),
          pltpu.VMEM((CH, 128), jnp.float32),
          pltpu.VMEM((CH, 128), jnp.float32),
          pltpu.VMEM((3 * ADIM, CH), jnp.float32),
          pltpu.VMEM((CH,), jnp.float32),
          pltpu.VMEM((CH,), jnp.float32),
          pltpu.VMEM((16,), jnp.float32),
          pltpu.VMEM((ZROWS, 128), jnp.float32),
          pltpu.VMEM_SHARED((NPAD, 128), jnp.float32),
          pltpu.SemaphoreType.DMA,
          pltpu.SemaphoreType.DMA,
          pltpu.SemaphoreType.DMA,
          pltpu.SemaphoreType.DMA,
      ],
  )
  def sc_kernel(sub_hbm, rel_hbm, ridx_hbm, obj_hbm, qrel_hbm,
                hidden_hbm, rela_hbm,
                s0, s1, s2, s3, s4, r0, r1, r2, r3, r4, q0, q1, q2, q3, q4,
                wb_hbm,
                msg_out, alpha_out, at_out, agg_out,
                sub_v, rel_v, ridx_v, obj_v, qr_v,
                hs_v, hr_v, att_v, al_v, at_v, wb_v, z_v, agg_sh,
                sem_idx, sem_gat, sem_qr, sem_wr):
    cid = lax.axis_index("c")
    sid = lax.axis_index("s")
    wid = sid * NC + cid
    scols = (s0, s1, s2, s3, s4)
    rcols = (r0, r1, r2, r3, r4)
    qcols = (q0, q1, q2, q3, q4)

    pltpu.sync_copy(wb_hbm, wb_v)

    def zbody(i, _):
      r = i // NV
      j = i % NV
      z_v[r, pl.ds(j * 16, 16)] = jnp.zeros((16,), jnp.float32)
      return 0
    lax.fori_loop(0, ZROWS * NV, zbody, 0)
    for j in range(NZ):
      pltpu.sync_copy(z_v, agg_sh.at[pl.ds(sid * ROWS_PER_SUB + j * ZROWS,
                                           ZROWS)])
    plsc.subcore_barrier()

    wv = wb_v[...]
    bias = wv[8]
    wk = (wv[0], wv[1], wv[2], wv[3], wv[4])

    def chunk_body(c, _):
      base = (wid + c * n_workers) * CH
      idx_cps = [
          pltpu.async_copy(sub_hbm.at[pl.ds(base, CH)], sub_v, sem_idx),
          pltpu.async_copy(rel_hbm.at[pl.ds(base, CH)], rel_v, sem_idx),
          pltpu.async_copy(ridx_hbm.at[pl.ds(base, CH)], ridx_v, sem_idx),
          pltpu.async_copy(obj_hbm.at[pl.ds(base, CH)], obj_v, sem_idx),
      ]
      for cp in idx_cps:
        cp.wait()
      pltpu.sync_copy(qrel_hbm.at[ridx_v], qr_v)
      g1 = [pltpu.async_copy(hidden_hbm.at[sub_v], hs_v, sem_gat),
            pltpu.async_copy(rela_hbm.at[rel_v], hr_v, sem_gat)]
      for k in range(ADIM):
        g1.append(pltpu.async_copy(scols[k].at[sub_v], att_v.at[k], sem_gat))
        g1.append(pltpu.async_copy(rcols[k].at[rel_v], att_v.at[ADIM + k],
                                   sem_gat))
        g1.append(pltpu.async_copy(qcols[k].at[qr_v], att_v.at[2 * ADIM + k],
                                   sem_gat))
      for cp in g1:
        cp.wait()

      for g in range(CH // 16):
        dg = pl.ds(g * 16, 16)
        acc = jnp.zeros((16,), jnp.float32)
        for k in range(ADIM):
          t = att_v[k, dg] + att_v[ADIM + k, dg] + att_v[2 * ADIM + k, dg]
          acc = acc + wk[k] * jnp.maximum(t, 0.0)
        at = acc + bias
        al = 1.0 / (1.0 + jnp.exp(-at))
        at_v[dg] = at
        al_v[dg] = al

      def mbody(g, _):
        a16 = al_v[pl.ds(g * 16, 16)]
        for i in range(16):
          e = g * 16 + i
          a = a16[i]
          for j in range(NV):
            hs = hs_v[e, pl.ds(j * 16, 16)]
            hr = hr_v[e, pl.ds(j * 16, 16)]
            hs_v[e, pl.ds(j * 16, 16)] = (hs + hr) * a
        return 0
      lax.fori_loop(0, CH // 16, mbody, 0)

      wr_cps = [
          pltpu.async_copy(hs_v, msg_out.at[pl.ds(base, CH)], sem_wr),
          pltpu.async_copy(al_v, alpha_out.at[pl.ds(base, CH)], sem_wr),
          pltpu.async_copy(at_v, at_out.at[pl.ds(base, CH)], sem_wr),
      ]
      pltpu.sync_copy(hs_v, agg_sh.at[obj_v], add=True)
      for cp in wr_cps:
        cp.wait()
      return 0
    n_chunks = TCH // n_workers + jnp.where(wid < TCH % n_workers, 1, 0)
    lax.fori_loop(0, n_chunks, chunk_body, 0)

    plsc.subcore_barrier()
    for j in range(NZ):
      rows = pl.ds(sid * ROWS_PER_SUB + j * ZROWS, ZROWS)
      pltpu.sync_copy(agg_sh.at[rows], agg_out.at[cid, rows])

  return sc_kernel


def kernel(q_sub, q_rel, hidden, edges, n_node, rela_embed, Ws_attn, Wr_attn,
           Wqr_attn, Wqr_bias, w_alpha, w_alpha_bias, W_h):
  N, D = hidden.shape
  V = rela_embed.shape[0]
  E = edges.shape[0]
  n_workers = 32

  sub = edges[:, 4].astype(jnp.int32)
  rel = edges[:, 2].astype(jnp.int32)
  obj = edges[:, 5].astype(jnp.int32)
  r_idx = edges[:, 0].astype(jnp.int32)
  q_rel32 = q_rel.astype(jnp.int32)

  def pad_w(w):
    return jnp.pad(w, ((0, 0), (0, ADIM_PAD - w.shape[1])))

  bias_pad = jnp.pad(Wqr_bias, (0, ADIM_PAD - Wqr_bias.shape[0]))[:, None]
  vpad = (-V) % 8
  rela_pad = jnp.pad(rela_embed, ((0, vpad), (0, 0)))
  Vp = V + vpad

  a_sT, a_rT, a_qT = pl.pallas_call(
      _precompute_body,
      out_shape=[
          jax.ShapeDtypeStruct((ADIM_PAD, N), jnp.float32),
          jax.ShapeDtypeStruct((ADIM_PAD, Vp), jnp.float32),
          jax.ShapeDtypeStruct((ADIM_PAD, Vp), jnp.float32),
      ],
  )(hidden, rela_pad, pad_w(Ws_attn), pad_w(Wr_attn), pad_w(Wqr_attn),
    bias_pad)

  wb = jnp.zeros((16,), jnp.float32)
  wb = wb.at[0:5].set(w_alpha[:, 0])
  wb = wb.at[8].set(w_alpha_bias[0])

  sc_kernel = _make_sc_kernel(E, N, D, n_workers)
  cols = ([a_sT[k] for k in range(ADIM)] + [a_rT[k] for k in range(ADIM)]
          + [a_qT[k] for k in range(ADIM)])
  message, alpha, alpha_temp, aggs = sc_kernel(
      sub, rel, r_idx, obj, q_rel32, hidden, rela_embed, *cols, wb)

  hidden_new = pl.pallas_call(
      _final_body,
      out_shape=jax.ShapeDtypeStruct((N, D), jnp.float32),
  )(aggs, W_h)

  return (hidden_new, alpha[:, None], message, obj, alpha_temp[:, None])


# speedup vs baseline: 4.7218x; 1.0634x over previous
import functools

import jax
import jax.numpy as jnp
from jax import lax
from jax.experimental import pallas as pl
from jax.experimental.pallas import tpu as pltpu
from jax.experimental.pallas import tpu_sc as plsc

ADIM = 5
ADIM_PAD = 8


def _precompute_body(hidden_ref, rela_ref, ws_ref, wr_ref, wq_ref, bias_ref,
                     as_ref, ar_ref, aq_ref):
  dn = (((0,), (1,)), ((), ()))
  as_ref[...] = lax.dot_general(ws_ref[...], hidden_ref[...], dn,
                                preferred_element_type=jnp.float32)
  ar_ref[...] = lax.dot_general(wr_ref[...], rela_ref[...], dn,
                                preferred_element_type=jnp.float32)
  aq_ref[...] = lax.dot_general(wq_ref[...], rela_ref[...], dn,
                                preferred_element_type=jnp.float32) + bias_ref[...]


def _final_body(agg_ref, wh_ref, out_ref):
  n = out_ref.shape[0]
  out_ref[...] = jnp.dot(agg_ref[0, :n, :] + agg_ref[1, :n, :], wh_ref[...],
                         preferred_element_type=jnp.float32)


def _make_sc_kernel(E, N, D, n_workers):
  CH = 128
  TCH = E // CH
  NC = 2
  NS = n_workers // NC
  NPAD = -(-N // (NS * 40)) * NS * 40
  ROWS_PER_SUB = NPAD // NS
  ZROWS = ROWS_PER_SUB // 10
  NZ = 10
  NV = D // 16

  mesh = plsc.VectorSubcoreMesh(core_axis_name="c", subcore_axis_name="s")

  @functools.partial(
      pl.kernel,
      mesh=mesh,
      out_type=[
          jax.ShapeDtypeStruct((E, D), jnp.float32),
          jax.ShapeDtypeStruct((E,), jnp.float32),
          jax.ShapeDtypeStruct((E,), jnp.float32),
          jax.ShapeDtypeStruct((NC, NPAD, D), jnp.float32),
      ],
      scratch_types=[
          pltpu.VMEM((CH,), jnp.int32),
          pltpu.VMEM((CH,), jnp.int32),
          pltpu.VMEM((CH,), jnp.int32),
          pltpu.VMEM((CH,), jnp.int32),
          pltpu.VMEM((CH,), jnp.int32),
          pltpu.VMEM((CH, 128), jnp.float32),
          pltpu.VMEM((CH, 128), jnp.float32),
          pltpu.VMEM((3 * ADIM, CH), jnp.float32),
          pltpu.VMEM((CH,), jnp.float32),
          pltpu.VMEM((CH,), jnp.float32),
          pltpu.VMEM((16,), jnp.float32),
          pltpu.VMEM((ZROWS, 128), jnp.float32),
          pltpu.VMEM_SHARED((NPAD, 128), jnp.float32),
          pltpu.SemaphoreType.DMA,
          pltpu.SemaphoreType.DMA,
          pltpu.SemaphoreType.DMA,
          pltpu.SemaphoreType.DMA,
      ],
  )
  def sc_kernel(sub_hbm, rel_hbm, ridx_hbm, obj_hbm, qrel_hbm,
                hidden_hbm, rela_hbm,
                s0, s1, s2, s3, s4, r0, r1, r2, r3, r4, q0, q1, q2, q3, q4,
                wb_hbm,
                msg_out, alpha_out, at_out, agg_out,
                sub_v, rel_v, ridx_v, obj_v, qr_v,
                hs_v, hr_v, att_v, al_v, at_v, wb_v, z_v, agg_sh,
                sem_idx, sem_gat, sem_qr, sem_wr):
    cid = lax.axis_index("c")
    sid = lax.axis_index("s")
    wid = sid * NC + cid
    scols = (s0, s1, s2, s3, s4)
    rcols = (r0, r1, r2, r3, r4)
    qcols = (q0, q1, q2, q3, q4)

    pltpu.sync_copy(wb_hbm, wb_v)

    def zbody(i, _):
      r = i // NV
      j = i % NV
      z_v[r, pl.ds(j * 16, 16)] = jnp.zeros((16,), jnp.float32)
      return 0
    lax.fori_loop(0, ZROWS * NV, zbody, 0)
    for j in range(NZ):
      pltpu.sync_copy(z_v, agg_sh.at[pl.ds(sid * ROWS_PER_SUB + j * ZROWS,
                                           ZROWS)])
    plsc.subcore_barrier()

    wv = wb_v[...]
    bias = wv[8]
    wk = (wv[0], wv[1], wv[2], wv[3], wv[4])

    def chunk_body(c, _):
      base = (wid + c * n_workers) * CH
      idx_cps = [
          pltpu.async_copy(sub_hbm.at[pl.ds(base, CH)], sub_v, sem_idx),
          pltpu.async_copy(rel_hbm.at[pl.ds(base, CH)], rel_v, sem_idx),
          pltpu.async_copy(ridx_hbm.at[pl.ds(base, CH)], ridx_v, sem_idx),
          pltpu.async_copy(obj_hbm.at[pl.ds(base, CH)], obj_v, sem_idx),
      ]
      for cp in idx_cps:
        cp.wait()
      g1 = [pltpu.async_copy(hidden_hbm.at[sub_v], hs_v, sem_gat),
            pltpu.async_copy(rela_hbm.at[rel_v], hr_v, sem_gat)]
      for k in range(ADIM):
        g1.append(pltpu.async_copy(scols[k].at[sub_v], att_v.at[k], sem_gat))
        g1.append(pltpu.async_copy(rcols[k].at[rel_v], att_v.at[ADIM + k],
                                   sem_gat))
      qr_cp = pltpu.async_copy(qrel_hbm.at[ridx_v], qr_v, sem_qr)
      qr_cp.wait()
      for k in range(ADIM):
        g1.append(pltpu.async_copy(qcols[k].at[qr_v], att_v.at[2 * ADIM + k],
                                   sem_gat))
      for cp in g1:
        cp.wait()

      for g in range(CH // 16):
        dg = pl.ds(g * 16, 16)
        acc = jnp.zeros((16,), jnp.float32)
        for k in range(ADIM):
          t = att_v[k, dg] + att_v[ADIM + k, dg] + att_v[2 * ADIM + k, dg]
          acc = acc + wk[k] * jnp.maximum(t, 0.0)
        at = acc + bias
        al = 1.0 / (1.0 + jnp.exp(-at))
        at_v[dg] = at
        al_v[dg] = al

      def mbody(g, _):
        a16 = al_v[pl.ds(g * 16, 16)]
        for i in range(16):
          e = g * 16 + i
          a = a16[i]
          for j in range(NV):
            hs = hs_v[e, pl.ds(j * 16, 16)]
            hr = hr_v[e, pl.ds(j * 16, 16)]
            hs_v[e, pl.ds(j * 16, 16)] = (hs + hr) * a
        return 0
      lax.fori_loop(0, CH // 16, mbody, 0)

      wr_cps = [
          pltpu.async_copy(hs_v, msg_out.at[pl.ds(base, CH)], sem_wr),
          pltpu.async_copy(al_v, alpha_out.at[pl.ds(base, CH)], sem_wr),
          pltpu.async_copy(at_v, at_out.at[pl.ds(base, CH)], sem_wr),
      ]
      pltpu.sync_copy(hs_v, agg_sh.at[obj_v], add=True)
      for cp in wr_cps:
        cp.wait()
      return 0
    n_chunks = TCH // n_workers + jnp.where(wid < TCH % n_workers, 1, 0)
    lax.fori_loop(0, n_chunks, chunk_body, 0)

    plsc.subcore_barrier()
    for j in range(NZ):
      rows = pl.ds(sid * ROWS_PER_SUB + j * ZROWS, ZROWS)
      pltpu.sync_copy(agg_sh.at[rows], agg_out.at[cid, rows])

  return sc_kernel


def kernel(q_sub, q_rel, hidden, edges, n_node, rela_embed, Ws_attn, Wr_attn,
           Wqr_attn, Wqr_bias, w_alpha, w_alpha_bias, W_h):
  N, D = hidden.shape
  V = rela_embed.shape[0]
  E = edges.shape[0]
  n_workers = 32

  sub = edges[:, 4].astype(jnp.int32)
  rel = edges[:, 2].astype(jnp.int32)
  obj = edges[:, 5].astype(jnp.int32)
  r_idx = edges[:, 0].astype(jnp.int32)
  q_rel32 = q_rel.astype(jnp.int32)

  def pad_w(w):
    return jnp.pad(w, ((0, 0), (0, ADIM_PAD - w.shape[1])))

  bias_pad = jnp.pad(Wqr_bias, (0, ADIM_PAD - Wqr_bias.shape[0]))[:, None]
  vpad = (-V) % 8
  rela_pad = jnp.pad(rela_embed, ((0, vpad), (0, 0)))
  Vp = V + vpad

  a_sT, a_rT, a_qT = pl.pallas_call(
      _precompute_body,
      out_shape=[
          jax.ShapeDtypeStruct((ADIM_PAD, N), jnp.float32),
          jax.ShapeDtypeStruct((ADIM_PAD, Vp), jnp.float32),
          jax.ShapeDtypeStruct((ADIM_PAD, Vp), jnp.float32),
      ],
  )(hidden, rela_pad, pad_w(Ws_attn), pad_w(Wr_attn), pad_w(Wqr_attn),
    bias_pad)

  wb = jnp.zeros((16,), jnp.float32)
  wb = wb.at[0:5].set(w_alpha[:, 0])
  wb = wb.at[8].set(w_alpha_bias[0])

  sc_kernel = _make_sc_kernel(E, N, D, n_workers)
  cols = ([a_sT[k] for k in range(ADIM)] + [a_rT[k] for k in range(ADIM)]
          + [a_qT[k] for k in range(ADIM)])
  message, alpha, alpha_temp, aggs = sc_kernel(
      sub, rel, r_idx, obj, q_rel32, hidden, rela_embed, *cols, wb)

  hidden_new = pl.pallas_call(
      _final_body,
      out_shape=jax.ShapeDtypeStruct((N, D), jnp.float32),
  )(aggs, W_h)

  return (hidden_new, alpha[:, None], message, obj, alpha_temp[:, None])


# speedup vs baseline: 4.8947x; 1.0366x over previous
import functools

import jax
import jax.numpy as jnp
from jax import lax
from jax.experimental import pallas as pl
from jax.experimental.pallas import tpu as pltpu
from jax.experimental.pallas import tpu_sc as plsc

ADIM = 5
ADIM_PAD = 8


def _precompute_body(hidden_ref, rela_ref, ws_ref, wr_ref, wq_ref, bias_ref,
                     as_ref, ar_ref, aq_ref):
  dn = (((0,), (1,)), ((), ()))
  as_ref[...] = lax.dot_general(ws_ref[...], hidden_ref[...], dn,
                                preferred_element_type=jnp.float32)
  ar_ref[...] = lax.dot_general(wr_ref[...], rela_ref[...], dn,
                                preferred_element_type=jnp.float32)
  aq_ref[...] = lax.dot_general(wq_ref[...], rela_ref[...], dn,
                                preferred_element_type=jnp.float32) + bias_ref[...]


def _final_body(agg_ref, wh_ref, out_ref):
  n = out_ref.shape[0]
  out_ref[...] = jnp.dot(agg_ref[0, :n, :] + agg_ref[1, :n, :], wh_ref[...],
                         preferred_element_type=jnp.float32)


def _make_sc_kernel(E, N, D, n_workers):
  CH = 128
  TCH = E // CH
  NC = 2
  NS = n_workers // NC
  NPAD = -(-N // (NS * 40)) * NS * 40
  ROWS_PER_SUB = NPAD // NS
  ZROWS = ROWS_PER_SUB // 10
  NZ = 10
  NV = D // 16

  mesh = plsc.VectorSubcoreMesh(core_axis_name="c", subcore_axis_name="s")

  @functools.partial(
      pl.kernel,
      mesh=mesh,
      out_type=[
          jax.ShapeDtypeStruct((E, D), jnp.float32),
          jax.ShapeDtypeStruct((E,), jnp.float32),
          jax.ShapeDtypeStruct((E,), jnp.float32),
          jax.ShapeDtypeStruct((NC, NPAD, D), jnp.float32),
      ],
      scratch_types=[
          pltpu.VMEM((CH,), jnp.int32),
          pltpu.VMEM((CH,), jnp.int32),
          pltpu.VMEM((CH,), jnp.int32),
          pltpu.VMEM((CH,), jnp.int32),
          pltpu.VMEM((CH,), jnp.int32),
          pltpu.VMEM((CH, 128), jnp.float32),
          pltpu.VMEM((CH, 128), jnp.float32),
          pltpu.VMEM((3 * ADIM, CH), jnp.float32),
          pltpu.VMEM((CH,), jnp.float32),
          pltpu.VMEM((CH,), jnp.float32),
          pltpu.VMEM((16,), jnp.float32),
          pltpu.VMEM((ZROWS, 128), jnp.float32),
          pltpu.VMEM_SHARED((NPAD, 128), jnp.float32),
          pltpu.SemaphoreType.DMA,
          pltpu.SemaphoreType.DMA,
          pltpu.SemaphoreType.DMA,
          pltpu.SemaphoreType.DMA,
      ],
  )
  def sc_kernel(sub_hbm, rel_hbm, ridx_hbm, obj_hbm, qrel_hbm,
                hidden_hbm, rela_hbm,
                s0, s1, s2, s3, s4, r0, r1, r2, r3, r4, q0, q1, q2, q3, q4,
                wb_hbm,
                msg_out, alpha_out, at_out, agg_out,
                sub_v, rel_v, ridx_v, obj_v, qr_v,
                hs_v, hr_v, att_v, al_v, at_v, wb_v, z_v, agg_sh,
                sem_idx, sem_gat, sem_qr, sem_wr):
    cid = lax.axis_index("c")
    sid = lax.axis_index("s")
    wid = sid * NC + cid
    scols = (s0, s1, s2, s3, s4)
    rcols = (r0, r1, r2, r3, r4)
    qcols = (q0, q1, q2, q3, q4)

    pltpu.sync_copy(wb_hbm, wb_v)

    def zbody(i, _):
      r = i // NV
      j = i % NV
      z_v[r, pl.ds(j * 16, 16)] = jnp.zeros((16,), jnp.float32)
      return 0
    lax.fori_loop(0, ZROWS * NV, zbody, 0)
    for j in range(NZ):
      pltpu.sync_copy(z_v, agg_sh.at[pl.ds(sid * ROWS_PER_SUB + j * ZROWS,
                                           ZROWS)])
    plsc.subcore_barrier()

    wv = wb_v[...]
    bias = wv[8]
    wk = (wv[0], wv[1], wv[2], wv[3], wv[4])

    n_chunks = TCH // n_workers + jnp.where(wid < TCH % n_workers, 1, 0)

    def idx_base(c):
      return (wid + c * n_workers) * CH

    b0 = idx_base(0)
    pltpu.async_copy(sub_hbm.at[pl.ds(b0, CH)], sub_v, sem_idx)
    pltpu.async_copy(rel_hbm.at[pl.ds(b0, CH)], rel_v, sem_idx)
    pltpu.async_copy(ridx_hbm.at[pl.ds(b0, CH)], ridx_v, sem_idx)
    pltpu.async_copy(obj_hbm.at[pl.ds(b0, CH)], obj_v, sem_idx)

    def drain_idx_desc():
      pltpu.make_async_copy(sub_hbm.at[pl.ds(0, CH)], sub_v, sem_idx).wait()
      pltpu.make_async_copy(rel_hbm.at[pl.ds(0, CH)], rel_v, sem_idx).wait()
      pltpu.make_async_copy(ridx_hbm.at[pl.ds(0, CH)], ridx_v,
                            sem_idx).wait()
      pltpu.make_async_copy(obj_hbm.at[pl.ds(0, CH)], obj_v, sem_idx).wait()

    def chunk_body(c, _):
      base = idx_base(c)
      drain_idx_desc()
      nbase = idx_base(jnp.where(c + 1 < n_chunks, c + 1, 0))
      g1 = [pltpu.async_copy(hidden_hbm.at[sub_v], hs_v, sem_gat),
            pltpu.async_copy(rela_hbm.at[rel_v], hr_v, sem_gat)]
      for k in range(ADIM):
        g1.append(pltpu.async_copy(scols[k].at[sub_v], att_v.at[k], sem_gat))
        g1.append(pltpu.async_copy(rcols[k].at[rel_v], att_v.at[ADIM + k],
                                   sem_gat))
      qr_cp = pltpu.async_copy(qrel_hbm.at[ridx_v], qr_v, sem_qr)
      qr_cp.wait()
      for k in range(ADIM):
        g1.append(pltpu.async_copy(qcols[k].at[qr_v], att_v.at[2 * ADIM + k],
                                   sem_gat))
      for cp in g1:
        cp.wait()
      pltpu.async_copy(sub_hbm.at[pl.ds(nbase, CH)], sub_v, sem_idx)
      pltpu.async_copy(rel_hbm.at[pl.ds(nbase, CH)], rel_v, sem_idx)
      pltpu.async_copy(ridx_hbm.at[pl.ds(nbase, CH)], ridx_v, sem_idx)

      for g in range(CH // 16):
        dg = pl.ds(g * 16, 16)
        acc = jnp.zeros((16,), jnp.float32)
        for k in range(ADIM):
          t = att_v[k, dg] + att_v[ADIM + k, dg] + att_v[2 * ADIM + k, dg]
          acc = acc + wk[k] * jnp.maximum(t, 0.0)
        at = acc + bias
        al = 1.0 / (1.0 + jnp.exp(-at))
        at_v[dg] = at
        al_v[dg] = al

      def mbody(g, _):
        a16 = al_v[pl.ds(g * 16, 16)]
        for i in range(16):
          e = g * 16 + i
          a = a16[i]
          for j in range(NV):
            hs = hs_v[e, pl.ds(j * 16, 16)]
            hr = hr_v[e, pl.ds(j * 16, 16)]
            hs_v[e, pl.ds(j * 16, 16)] = (hs + hr) * a
        return 0
      lax.fori_loop(0, CH // 16, mbody, 0)

      wr_cps = [
          pltpu.async_copy(hs_v, msg_out.at[pl.ds(base, CH)], sem_wr),
          pltpu.async_copy(al_v, alpha_out.at[pl.ds(base, CH)], sem_wr),
          pltpu.async_copy(at_v, at_out.at[pl.ds(base, CH)], sem_wr),
      ]
      pltpu.sync_copy(hs_v, agg_sh.at[obj_v], add=True)
      pltpu.async_copy(obj_hbm.at[pl.ds(nbase, CH)], obj_v, sem_idx)
      for cp in wr_cps:
        cp.wait()
      return 0
    lax.fori_loop(0, n_chunks, chunk_body, 0)
    drain_idx_desc()

    plsc.subcore_barrier()
    for j in range(NZ):
      rows = pl.ds(sid * ROWS_PER_SUB + j * ZROWS, ZROWS)
      pltpu.sync_copy(agg_sh.at[rows], agg_out.at[cid, rows])

  return sc_kernel


def kernel(q_sub, q_rel, hidden, edges, n_node, rela_embed, Ws_attn, Wr_attn,
           Wqr_attn, Wqr_bias, w_alpha, w_alpha_bias, W_h):
  N, D = hidden.shape
  V = rela_embed.shape[0]
  E = edges.shape[0]
  n_workers = 32

  sub = edges[:, 4].astype(jnp.int32)
  rel = edges[:, 2].astype(jnp.int32)
  obj = edges[:, 5].astype(jnp.int32)
  r_idx = edges[:, 0].astype(jnp.int32)
  q_rel32 = q_rel.astype(jnp.int32)

  def pad_w(w):
    return jnp.pad(w, ((0, 0), (0, ADIM_PAD - w.shape[1])))

  bias_pad = jnp.pad(Wqr_bias, (0, ADIM_PAD - Wqr_bias.shape[0]))[:, None]
  vpad = (-V) % 8
  rela_pad = jnp.pad(rela_embed, ((0, vpad), (0, 0)))
  Vp = V + vpad

  a_sT, a_rT, a_qT = pl.pallas_call(
      _precompute_body,
      out_shape=[
          jax.ShapeDtypeStruct((ADIM_PAD, N), jnp.float32),
          jax.ShapeDtypeStruct((ADIM_PAD, Vp), jnp.float32),
          jax.ShapeDtypeStruct((ADIM_PAD, Vp), jnp.float32),
      ],
  )(hidden, rela_pad, pad_w(Ws_attn), pad_w(Wr_attn), pad_w(Wqr_attn),
    bias_pad)

  wb = jnp.zeros((16,), jnp.float32)
  wb = wb.at[0:5].set(w_alpha[:, 0])
  wb = wb.at[8].set(w_alpha_bias[0])

  sc_kernel = _make_sc_kernel(E, N, D, n_workers)
  cols = ([a_sT[k] for k in range(ADIM)] + [a_rT[k] for k in range(ADIM)]
          + [a_qT[k] for k in range(ADIM)])
  message, alpha, alpha_temp, aggs = sc_kernel(
      sub, rel, r_idx, obj, q_rel32, hidden, rela_embed, *cols, wb)

  hidden_new = pl.pallas_call(
      _final_body,
      out_shape=jax.ShapeDtypeStruct((N, D), jnp.float32),
  )(aggs, W_h)

  return (hidden_new, alpha[:, None], message, obj, alpha_temp[:, None])


# speedup vs baseline: 5.1023x; 1.0424x over previous
import functools

import jax
import jax.numpy as jnp
from jax import lax
from jax.experimental import pallas as pl
from jax.experimental.pallas import tpu as pltpu
from jax.experimental.pallas import tpu_sc as plsc

ADIM = 5
ADIM_PAD = 8


def _precompute_body(hidden_ref, rela_ref, ws_ref, wr_ref, wq_ref, bias_ref,
                     as_ref, ar_ref, aq_ref):
  dn = (((0,), (1,)), ((), ()))
  as_ref[...] = lax.dot_general(ws_ref[...], hidden_ref[...], dn,
                                preferred_element_type=jnp.float32)
  ar_ref[...] = lax.dot_general(wr_ref[...], rela_ref[...], dn,
                                preferred_element_type=jnp.float32)
  aq_ref[...] = lax.dot_general(wq_ref[...], rela_ref[...], dn,
                                preferred_element_type=jnp.float32) + bias_ref[...]


def _final_body(agg_ref, wh_ref, out_ref):
  n = out_ref.shape[0]
  out_ref[...] = jnp.dot(agg_ref[0, :n, :] + agg_ref[1, :n, :], wh_ref[...],
                         preferred_element_type=jnp.float32)


def _make_sc_kernel(E, N, D, n_workers):
  CH = 128
  TCH = E // CH
  NC = 2
  NS = n_workers // NC
  NPAD = -(-N // (NS * 40)) * NS * 40
  ROWS_PER_SUB = NPAD // NS
  ZROWS = ROWS_PER_SUB // 10
  NZ = 10
  NV = D // 16

  mesh = plsc.VectorSubcoreMesh(core_axis_name="c", subcore_axis_name="s")

  @functools.partial(
      pl.kernel,
      mesh=mesh,
      out_type=[
          jax.ShapeDtypeStruct((E, D), jnp.float32),
          jax.ShapeDtypeStruct((E,), jnp.float32),
          jax.ShapeDtypeStruct((E,), jnp.float32),
          jax.ShapeDtypeStruct((NC, NPAD, D), jnp.float32),
      ],
      scratch_types=[
          pltpu.VMEM((CH,), jnp.int32),
          pltpu.VMEM((CH,), jnp.int32),
          pltpu.VMEM((CH,), jnp.int32),
          pltpu.VMEM((CH,), jnp.int32),
          pltpu.VMEM((CH,), jnp.int32),
          pltpu.VMEM((CH, 128), jnp.float32),
          pltpu.VMEM((CH, 128), jnp.float32),
          pltpu.VMEM((3 * ADIM, CH), jnp.float32),
          pltpu.VMEM((CH,), jnp.float32),
          pltpu.VMEM((CH,), jnp.float32),
          pltpu.VMEM((16,), jnp.float32),
          pltpu.VMEM((ZROWS, 128), jnp.float32),
          pltpu.VMEM_SHARED((NPAD, 128), jnp.float32),
          pltpu.SemaphoreType.DMA,
          pltpu.SemaphoreType.DMA,
          pltpu.SemaphoreType.DMA,
          pltpu.SemaphoreType.DMA,
          pltpu.SemaphoreType.DMA,
      ],
  )
  def sc_kernel(sub_hbm, rel_hbm, ridx_hbm, obj_hbm, qrel_hbm,
                hidden_hbm, rela_hbm,
                s0, s1, s2, s3, s4, r0, r1, r2, r3, r4, q0, q1, q2, q3, q4,
                wb_hbm,
                msg_out, alpha_out, at_out, agg_out,
                sub_v, rel_v, ridx_v, obj_v, qr_v,
                hs_v, hr_v, att_v, al_v, at_v, wb_v, z_v, agg_sh,
                sem_idx, sem_gat, sem_qr, sem_wr, sem_obj):
    cid = lax.axis_index("c")
    sid = lax.axis_index("s")
    wid = sid * NC + cid
    scols = (s0, s1, s2, s3, s4)
    rcols = (r0, r1, r2, r3, r4)
    qcols = (q0, q1, q2, q3, q4)

    pltpu.sync_copy(wb_hbm, wb_v)

    def zbody(i, _):
      r = i // NV
      j = i % NV
      z_v[r, pl.ds(j * 16, 16)] = jnp.zeros((16,), jnp.float32)
      return 0
    lax.fori_loop(0, ZROWS * NV, zbody, 0)
    for j in range(NZ):
      pltpu.sync_copy(z_v, agg_sh.at[pl.ds(sid * ROWS_PER_SUB + j * ZROWS,
                                           ZROWS)])
    plsc.subcore_barrier()

    wv = wb_v[...]
    bias = wv[8]
    wk = (wv[0], wv[1], wv[2], wv[3], wv[4])

    n_chunks = TCH // n_workers + jnp.where(wid < TCH % n_workers, 1, 0)

    def idx_base(c):
      return (wid + c * n_workers) * CH

    def drain_idx3_desc():
      pltpu.make_async_copy(sub_hbm.at[pl.ds(0, CH)], sub_v, sem_idx).wait()
      pltpu.make_async_copy(rel_hbm.at[pl.ds(0, CH)], rel_v, sem_idx).wait()
      pltpu.make_async_copy(ridx_hbm.at[pl.ds(0, CH)], ridx_v,
                            sem_idx).wait()

    def drain_obj_desc():
      pltpu.make_async_copy(obj_hbm.at[pl.ds(0, CH)], obj_v, sem_obj).wait()

    def drain_qr_desc():
      pltpu.make_async_copy(qrel_hbm.at[pl.ds(0, CH)], qr_v, sem_qr).wait()

    b0 = idx_base(0)
    pltpu.async_copy(sub_hbm.at[pl.ds(b0, CH)], sub_v, sem_idx)
    pltpu.async_copy(rel_hbm.at[pl.ds(b0, CH)], rel_v, sem_idx)
    pltpu.async_copy(ridx_hbm.at[pl.ds(b0, CH)], ridx_v, sem_idx)
    pltpu.async_copy(obj_hbm.at[pl.ds(b0, CH)], obj_v, sem_obj)
    drain_idx3_desc()
    pltpu.async_copy(qrel_hbm.at[ridx_v], qr_v, sem_qr)

    def chunk_body(c, _):
      base = idx_base(c)
      nbase = idx_base(jnp.where(c + 1 < n_chunks, c + 1, 0))
      g1 = [pltpu.async_copy(hidden_hbm.at[sub_v], hs_v, sem_gat),
            pltpu.async_copy(rela_hbm.at[rel_v], hr_v, sem_gat)]
      for k in range(ADIM):
        g1.append(pltpu.async_copy(scols[k].at[sub_v], att_v.at[k], sem_gat))
        g1.append(pltpu.async_copy(rcols[k].at[rel_v], att_v.at[ADIM + k],
                                   sem_gat))
      drain_qr_desc()
      for k in range(ADIM):
        g1.append(pltpu.async_copy(qcols[k].at[qr_v], att_v.at[2 * ADIM + k],
                                   sem_gat))
      for cp in g1:
        cp.wait()
      pltpu.async_copy(sub_hbm.at[pl.ds(nbase, CH)], sub_v, sem_idx)
      pltpu.async_copy(rel_hbm.at[pl.ds(nbase, CH)], rel_v, sem_idx)
      pltpu.async_copy(ridx_hbm.at[pl.ds(nbase, CH)], ridx_v, sem_idx)

      for g in range(CH // 16):
        dg = pl.ds(g * 16, 16)
        acc = jnp.zeros((16,), jnp.float32)
        for k in range(ADIM):
          t = att_v[k, dg] + att_v[ADIM + k, dg] + att_v[2 * ADIM + k, dg]
          acc = acc + wk[k] * jnp.maximum(t, 0.0)
        at = acc + bias
        al = 1.0 / (1.0 + jnp.exp(-at))
        at_v[dg] = at
        al_v[dg] = al

      def mbody(g, _):
        a16 = al_v[pl.ds(g * 16, 16)]
        for i in range(16):
          e = g * 16 + i
          a = a16[i]
          for j in range(NV):
            hs = hs_v[e, pl.ds(j * 16, 16)]
            hr = hr_v[e, pl.ds(j * 16, 16)]
            hs_v[e, pl.ds(j * 16, 16)] = (hs + hr) * a
        return 0
      lax.fori_loop(0, CH // 16, mbody, 0)

      wr_cps = [
          pltpu.async_copy(hs_v, msg_out.at[pl.ds(base, CH)], sem_wr),
          pltpu.async_copy(al_v, alpha_out.at[pl.ds(base, CH)], sem_wr),
          pltpu.async_copy(at_v, at_out.at[pl.ds(base, CH)], sem_wr),
      ]
      drain_obj_desc()
      pltpu.sync_copy(hs_v, agg_sh.at[obj_v], add=True)
      pltpu.async_copy(obj_hbm.at[pl.ds(nbase, CH)], obj_v, sem_obj)
      drain_idx3_desc()
      pltpu.async_copy(qrel_hbm.at[ridx_v], qr_v, sem_qr)
      for cp in wr_cps:
        cp.wait()
      return 0
    lax.fori_loop(0, n_chunks, chunk_body, 0)
    drain_obj_desc()
    drain_qr_desc()

    plsc.subcore_barrier()
    for j in range(NZ):
      rows = pl.ds(sid * ROWS_PER_SUB + j * ZROWS, ZROWS)
      pltpu.sync_copy(agg_sh.at[rows], agg_out.at[cid, rows])

  return sc_kernel


def kernel(q_sub, q_rel, hidden, edges, n_node, rela_embed, Ws_attn, Wr_attn,
           Wqr_attn, Wqr_bias, w_alpha, w_alpha_bias, W_h):
  N, D = hidden.shape
  V = rela_embed.shape[0]
  E = edges.shape[0]
  n_workers = 32

  sub = edges[:, 4].astype(jnp.int32)
  rel = edges[:, 2].astype(jnp.int32)
  obj = edges[:, 5].astype(jnp.int32)
  r_idx = edges[:, 0].astype(jnp.int32)
  q_rel32 = q_rel.astype(jnp.int32)

  def pad_w(w):
    return jnp.pad(w, ((0, 0), (0, ADIM_PAD - w.shape[1])))

  bias_pad = jnp.pad(Wqr_bias, (0, ADIM_PAD - Wqr_bias.shape[0]))[:, None]
  vpad = (-V) % 8
  rela_pad = jnp.pad(rela_embed, ((0, vpad), (0, 0)))
  Vp = V + vpad

  a_sT, a_rT, a_qT = pl.pallas_call(
      _precompute_body,
      out_shape=[
          jax.ShapeDtypeStruct((ADIM_PAD, N), jnp.float32),
          jax.ShapeDtypeStruct((ADIM_PAD, Vp), jnp.float32),
          jax.ShapeDtypeStruct((ADIM_PAD, Vp), jnp.float32),
      ],
  )(hidden, rela_pad, pad_w(Ws_attn), pad_w(Wr_attn), pad_w(Wqr_attn),
    bias_pad)

  wb = jnp.zeros((16,), jnp.float32)
  wb = wb.at[0:5].set(w_alpha[:, 0])
  wb = wb.at[8].set(w_alpha_bias[0])

  sc_kernel = _make_sc_kernel(E, N, D, n_workers)
  cols = ([a_sT[k] for k in range(ADIM)] + [a_rT[k] for k in range(ADIM)]
          + [a_qT[k] for k in range(ADIM)])
  message, alpha, alpha_temp, aggs = sc_kernel(
      sub, rel, r_idx, obj, q_rel32, hidden, rela_embed, *cols, wb)

  hidden_new = pl.pallas_call(
      _final_body,
      out_shape=jax.ShapeDtypeStruct((N, D), jnp.float32),
  )(aggs, W_h)

  return (hidden_new, alpha[:, None], message, obj, alpha_temp[:, None])
